# baseline (speedup 1.0000x reference)
import functools

import jax
import jax.numpy as jnp
from jax import lax
from jax.experimental import pallas as pl
from jax.experimental.pallas import tpu as pltpu
from jax.experimental.pallas import tpu_sc as plsc

NC = 2
NS = 16
NW = NC * NS
L = 16

CHUNK = 128


def _wid():
    return lax.axis_index("s") * NC + lax.axis_index("c")


def _sc_analyze(M, B, R, idx_hbm, wrow_hbm, wpos_hbm, counts_hbm,
                idx_v, row_buf, pos_buf, table, rot16, keep16, cnt16):
    wid = _wid()
    lo = wid * R
    hi = jnp.minimum(lo + R, M)
    iota = lax.iota(jnp.int32, L)

    pltpu.sync_copy(idx_hbm, idx_v)

    def scan_body(i, cursor):
        x = plsc.load_gather(idx_v, [i * L + iota])
        pos = i * L + iota
        m = (x >= lo) & (x < hi)
        mi = m.astype(jnp.int32)
        cnt = jnp.sum(mi)

        @pl.when(cnt > 0)
        def _():
            dest = cursor + jnp.cumsum(mi) - 1
            plsc.store_scatter(
                row_buf, [dest >> 7, dest & (CHUNK - 1)], x, mask=m)
            plsc.store_scatter(
                pos_buf, [dest >> 7, dest & (CHUNK - 1)], pos, mask=m)
            keep16[...] = mi

            @pl.when(cnt > 1)
            def _():
                rot16[...] = jnp.where(m, x, -1)
                dup = jnp.zeros((L,), jnp.bool_)
                for r in range(1, L):
                    y = plsc.load_gather(rot16, [(iota + r) & (L - 1)])
                    later = (iota + r) < L
                    dup = dup | (m & later & (y == x))
                keep16[...] = jnp.where(dup, 0, mi)

            keep = keep16[...] > 0
            plsc.store_scatter(table, [x - lo], pos, mask=keep)

        return cursor + cnt

    n_cand = lax.fori_loop(0, B // L, scan_body, jnp.int32(0))

    def win_body(c, carry):
        wcur, pad_row, pad_pos = carry
        fl = c * L + iota
        valid = fl < n_cand
        fb, fc = fl >> 7, fl & (CHUNK - 1)
        x = plsc.load_gather(row_buf, [fb, fc], mask=valid)
        p = plsc.load_gather(pos_buf, [fb, fc], mask=valid)
        w = plsc.load_gather(table, [jnp.where(valid, x - lo, 0)], mask=valid)
        keep = valid & (w == p)
        ki = keep.astype(jnp.int32)
        kcnt = jnp.sum(ki)
        dest = wcur + jnp.cumsum(ki) - 1
        plsc.store_scatter(row_buf, [dest >> 7, dest & (CHUNK - 1)], x, mask=keep)
        plsc.store_scatter(pos_buf, [dest >> 7, dest & (CHUNK - 1)], p, mask=keep)
        lmax = jnp.max(jnp.where(keep, iota, -1))
        sel = keep & (iota == lmax)
        pr = jnp.max(jnp.where(sel, x, -1))
        pp = jnp.max(jnp.where(sel, p, -1))
        pad_row = jnp.where(kcnt > 0, pr, pad_row)
        pad_pos = jnp.where(kcnt > 0, pp, pad_pos)
        return wcur + kcnt, pad_row, pad_pos

    n_win, pad_row, pad_pos = lax.fori_loop(
        0, pl.cdiv(n_cand, L), win_body,
        (jnp.int32(0), jnp.int32(0), jnp.int32(0)))

    n_tot = pl.cdiv(n_win, CHUNK) * CHUNK

    def pad_body(c, _):
        e = n_win + c * L + iota
        mm = e < n_tot
        plsc.store_scatter(
            row_buf, [e >> 7, e & (CHUNK - 1)],
            jnp.full((L,), pad_row, jnp.int32), mask=mm)
        plsc.store_scatter(
            pos_buf, [e >> 7, e & (CHUNK - 1)],
            jnp.full((L,), pad_pos, jnp.int32), mask=mm)
        return 0

    lax.fori_loop(0, pl.cdiv(n_tot - n_win, L), pad_body, 0)

    def out_body(b, _):
        pltpu.sync_copy(row_buf.at[b], wrow_hbm.at[wid, b])
        pltpu.sync_copy(pos_buf.at[b], wpos_hbm.at[wid, b])
        return 0

    lax.fori_loop(0, n_tot // CHUNK, out_body, 0)
    cnt16[...] = jnp.full((L,), n_tot, jnp.int32)
    pltpu.sync_copy(cnt16, counts_hbm.at[wid])


def _sc_apply(wrow_hbm, wpos_hbm, counts_hbm, val_hbm, out_hbm,
              pos_st, row_st, cnt16, rows0, rows1,
              sem_g0, sem_g1, sem_s0, sem_s1):
    wid = _wid()
    pltpu.sync_copy(counts_hbm.at[wid], cnt16)
    n_tot = jnp.max(cnt16[...])
    nb = n_tot // CHUNK

    def stage(b, slot):
        pltpu.sync_copy(wpos_hbm.at[wid, b], pos_st.at[slot])
        pltpu.sync_copy(wrow_hbm.at[wid, b], row_st.at[slot])

    def gather(b, slot, rows, sem):
        stage(b, slot)
        pltpu.make_async_copy(
            val_hbm.at[pos_st.at[slot]], rows, sem).start()

    @pl.when(nb > 0)
    def _():
        gather(0, 0, rows0, sem_g0)

    def pair_body(i, _):
        b = i * 2
        pltpu.make_async_copy(val_hbm.at[pos_st.at[0]], rows0, sem_g0).wait()
        s0 = pltpu.make_async_copy(rows0, out_hbm.at[row_st.at[0]], sem_s0)
        s0.start()

        @pl.when(b + 1 < nb)
        def _():
            gather(b + 1, 1, rows1, sem_g1)

        s0.wait()

        @pl.when(b + 1 < nb)
        def _():
            pltpu.make_async_copy(
                val_hbm.at[pos_st.at[1]], rows1, sem_g1).wait()
            s1 = pltpu.make_async_copy(
                rows1, out_hbm.at[row_st.at[1]], sem_s1)
            s1.start()

            @pl.when(b + 2 < nb)
            def _():
                gather(b + 2, 0, rows0, sem_g0)

            s1.wait()

        return 0

    lax.fori_loop(0, pl.cdiv(nb, 2), pair_body, 0)


def kernel(mem, idx, val):
    M, D = mem.shape
    B, _ = val.shape
    assert B % L == 0
    R = (M + NW - 1) // NW
    assert 0 < M - (NW - 1) * R <= R
    NB_MAX = B // CHUNK

    mesh = plsc.VectorSubcoreMesh(
        core_axis_name="c", subcore_axis_name="s", num_cores=NC)
    params = pltpu.CompilerParams(needs_layout_passes=False)

    analyze = pl.kernel(
        functools.partial(_sc_analyze, M, B, R),
        out_type=(
            jax.ShapeDtypeStruct((NW, NB_MAX, CHUNK), jnp.int32),
            jax.ShapeDtypeStruct((NW, NB_MAX, CHUNK), jnp.int32),
            jax.ShapeDtypeStruct((NW, L), jnp.int32),
        ),
        mesh=mesh,
        compiler_params=params,
        cost_estimate=pl.CostEstimate(
            flops=0, bytes_accessed=120_000_000, transcendentals=0),
        scratch_types=[
            pltpu.VMEM((B,), jnp.int32),
            pltpu.VMEM((NB_MAX, CHUNK), jnp.int32),
            pltpu.VMEM((NB_MAX, CHUNK), jnp.int32),
            pltpu.VMEM((R,), jnp.int32),
            pltpu.VMEM((L,), jnp.int32),
            pltpu.VMEM((L,), jnp.int32),
            pltpu.VMEM((L,), jnp.int32),
        ],
    )

    apply_k = pl.kernel(
        _sc_apply,
        out_type=(),
        mesh=mesh,
        compiler_params=params,
        cost_estimate=pl.CostEstimate(
            flops=0, bytes_accessed=80_000_000, transcendentals=0),
        scratch_types=[
            pltpu.VMEM((2, CHUNK), jnp.int32),
            pltpu.VMEM((2, CHUNK), jnp.int32),
            pltpu.VMEM((L,), jnp.int32),
            pltpu.VMEM((CHUNK, D), jnp.float32),
            pltpu.VMEM((CHUNK, D), jnp.float32),
            pltpu.SemaphoreType.DMA,
            pltpu.SemaphoreType.DMA,
            pltpu.SemaphoreType.DMA,
            pltpu.SemaphoreType.DMA,
        ],
    )

    wrow, wpos, counts = analyze(idx)
    out_ref = jax.new_ref(mem)
    apply_k(wrow, wpos, counts, val, out_ref)
    return out_ref[...]


# speedup vs baseline: 1.0279x; 1.0279x over previous
import functools

import jax
import jax.numpy as jnp
from jax import lax
from jax.experimental import pallas as pl
from jax.experimental.pallas import tpu as pltpu
from jax.experimental.pallas import tpu_sc as plsc

NC = 2
NS = 16
NW = NC * NS
L = 16

CHUNK = 128


def _wid():
    return lax.axis_index("s") * NC + lax.axis_index("c")


def _sc_analyze(M, B, R, idx_hbm, wrow_hbm, wpos_hbm, counts_hbm,
                idx_v, row_buf, pos_buf, table, rot16, keep16, cnt16):
    wid = _wid()
    lo = wid * R
    hi = jnp.minimum(lo + R, M)
    iota = lax.iota(jnp.int32, L)

    pltpu.sync_copy(idx_hbm, idx_v)

    def one_chunk(base, cursor):
        x = plsc.load_gather(idx_v, [base + iota])
        pos = base + iota
        m = (x >= lo) & (x < hi)
        mi = m.astype(jnp.int32)
        cnt = jnp.sum(mi)

        @pl.when(cnt > 0)
        def _():
            dest = cursor + jnp.cumsum(mi) - 1
            plsc.store_scatter(
                row_buf, [dest >> 7, dest & (CHUNK - 1)], x, mask=m)
            plsc.store_scatter(
                pos_buf, [dest >> 7, dest & (CHUNK - 1)], pos, mask=m)
            keep16[...] = mi

            @pl.when(cnt > 1)
            def _():
                rot16[...] = jnp.where(m, x, -1)
                dup = jnp.zeros((L,), jnp.bool_)
                for r in range(1, L):
                    y = plsc.load_gather(rot16, [(iota + r) & (L - 1)])
                    later = (iota + r) < L
                    dup = dup | (m & later & (y == x))
                keep16[...] = jnp.where(dup, 0, mi)

            keep = keep16[...] > 0
            plsc.store_scatter(table, [x - lo], pos, mask=keep)

        return cursor + cnt

    def scan_body(i, cursor):
        cursor = one_chunk(i * 2 * L, cursor)
        cursor = one_chunk(i * 2 * L + L, cursor)
        return cursor

    n_cand = lax.fori_loop(0, B // (2 * L), scan_body, jnp.int32(0))

    def win_body(c, carry):
        wcur, pad_row, pad_pos = carry
        fl = c * L + iota
        valid = fl < n_cand
        fb, fc = fl >> 7, fl & (CHUNK - 1)
        x = plsc.load_gather(row_buf, [fb, fc], mask=valid)
        p = plsc.load_gather(pos_buf, [fb, fc], mask=valid)
        w = plsc.load_gather(table, [jnp.where(valid, x - lo, 0)], mask=valid)
        keep = valid & (w == p)
        ki = keep.astype(jnp.int32)
        kcnt = jnp.sum(ki)
        dest = wcur + jnp.cumsum(ki) - 1
        plsc.store_scatter(row_buf, [dest >> 7, dest & (CHUNK - 1)], x, mask=keep)
        plsc.store_scatter(pos_buf, [dest >> 7, dest & (CHUNK - 1)], p, mask=keep)
        lmax = jnp.max(jnp.where(keep, iota, -1))
        sel = keep & (iota == lmax)
        pr = jnp.max(jnp.where(sel, x, -1))
        pp = jnp.max(jnp.where(sel, p, -1))
        pad_row = jnp.where(kcnt > 0, pr, pad_row)
        pad_pos = jnp.where(kcnt > 0, pp, pad_pos)
        return wcur + kcnt, pad_row, pad_pos

    n_win, pad_row, pad_pos = lax.fori_loop(
        0, pl.cdiv(n_cand, L), win_body,
        (jnp.int32(0), jnp.int32(0), jnp.int32(0)))

    n_tot = pl.cdiv(n_win, CHUNK) * CHUNK

    def pad_body(c, _):
        e = n_win + c * L + iota
        mm = e < n_tot
        plsc.store_scatter(
            row_buf, [e >> 7, e & (CHUNK - 1)],
            jnp.full((L,), pad_row, jnp.int32), mask=mm)
        plsc.store_scatter(
            pos_buf, [e >> 7, e & (CHUNK - 1)],
            jnp.full((L,), pad_pos, jnp.int32), mask=mm)
        return 0

    lax.fori_loop(0, pl.cdiv(n_tot - n_win, L), pad_body, 0)

    def out_body(b, _):
        pltpu.sync_copy(row_buf.at[b], wrow_hbm.at[wid, b])
        pltpu.sync_copy(pos_buf.at[b], wpos_hbm.at[wid, b])
        return 0

    lax.fori_loop(0, n_tot // CHUNK, out_body, 0)
    cnt16[...] = jnp.full((L,), n_tot, jnp.int32)
    pltpu.sync_copy(cnt16, counts_hbm.at[wid])


PRE = 16


def _sc_apply(wrow_hbm, wpos_hbm, counts_hbm, val_hbm, out_hbm,
              pos_st, row_st, cnt16, rows0, rows1,
              sem_pre, sem_g0, sem_g1, sem_s0, sem_s1):
    wid = _wid()
    pre_p = pltpu.make_async_copy(
        wpos_hbm.at[wid, pl.ds(0, PRE)], pos_st, sem_pre)
    pre_p.start()
    pre_r = pltpu.make_async_copy(
        wrow_hbm.at[wid, pl.ds(0, PRE)], row_st, sem_pre)
    pre_r.start()
    pltpu.sync_copy(counts_hbm.at[wid], cnt16)
    n_tot = jnp.max(cnt16[...])
    nb = n_tot // CHUNK
    pre_p.wait()
    pre_r.wait()

    def slot_of(b):
        return jnp.where(b < PRE, b, b & 1)

    def gather(b, rows, sem):
        @pl.when(b >= PRE)
        def _():
            pltpu.sync_copy(wpos_hbm.at[wid, b], pos_st.at[b & 1])
            pltpu.sync_copy(wrow_hbm.at[wid, b], row_st.at[b & 1])

        pltpu.make_async_copy(
            val_hbm.at[pos_st.at[slot_of(b)]], rows, sem).start()

    @pl.when(nb > 0)
    def _():
        gather(0, rows0, sem_g0)

    def pair_body(i, _):
        b = i * 2
        pltpu.make_async_copy(
            val_hbm.at[pos_st.at[slot_of(b)]], rows0, sem_g0).wait()
        s0 = pltpu.make_async_copy(
            rows0, out_hbm.at[row_st.at[slot_of(b)]], sem_s0)
        s0.start()

        @pl.when(b + 1 < nb)
        def _():
            gather(b + 1, rows1, sem_g1)

        s0.wait()

        @pl.when(b + 1 < nb)
        def _():
            pltpu.make_async_copy(
                val_hbm.at[pos_st.at[slot_of(b + 1)]], rows1, sem_g1).wait()
            s1 = pltpu.make_async_copy(
                rows1, out_hbm.at[row_st.at[slot_of(b + 1)]], sem_s1)
            s1.start()

            @pl.when(b + 2 < nb)
            def _():
                gather(b + 2, rows0, sem_g0)

            s1.wait()

        return 0

    lax.fori_loop(0, pl.cdiv(nb, 2), pair_body, 0)


def kernel(mem, idx, val):
    M, D = mem.shape
    B, _ = val.shape
    assert B % L == 0
    R = (M + NW - 1) // NW
    assert 0 < M - (NW - 1) * R <= R
    NB_MAX = B // CHUNK

    mesh = plsc.VectorSubcoreMesh(
        core_axis_name="c", subcore_axis_name="s", num_cores=NC)
    params = pltpu.CompilerParams(needs_layout_passes=False)

    analyze = pl.kernel(
        functools.partial(_sc_analyze, M, B, R),
        out_type=(
            jax.ShapeDtypeStruct((NW, NB_MAX, CHUNK), jnp.int32),
            jax.ShapeDtypeStruct((NW, NB_MAX, CHUNK), jnp.int32),
            jax.ShapeDtypeStruct((NW, L), jnp.int32),
        ),
        mesh=mesh,
        compiler_params=params,
        cost_estimate=pl.CostEstimate(
            flops=0, bytes_accessed=120_000_000, transcendentals=0),
        scratch_types=[
            pltpu.VMEM((B,), jnp.int32),
            pltpu.VMEM((NB_MAX, CHUNK), jnp.int32),
            pltpu.VMEM((NB_MAX, CHUNK), jnp.int32),
            pltpu.VMEM((R,), jnp.int32),
            pltpu.VMEM((L,), jnp.int32),
            pltpu.VMEM((L,), jnp.int32),
            pltpu.VMEM((L,), jnp.int32),
        ],
    )

    apply_k = pl.kernel(
        _sc_apply,
        out_type=(),
        mesh=mesh,
        compiler_params=params,
        cost_estimate=pl.CostEstimate(
            flops=0, bytes_accessed=80_000_000, transcendentals=0),
        scratch_types=[
            pltpu.VMEM((PRE, CHUNK), jnp.int32),
            pltpu.VMEM((PRE, CHUNK), jnp.int32),
            pltpu.VMEM((L,), jnp.int32),
            pltpu.VMEM((CHUNK, D), jnp.float32),
            pltpu.VMEM((CHUNK, D), jnp.float32),
            pltpu.SemaphoreType.DMA,
            pltpu.SemaphoreType.DMA,
            pltpu.SemaphoreType.DMA,
            pltpu.SemaphoreType.DMA,
            pltpu.SemaphoreType.DMA,
        ],
    )

    wrow, wpos, counts = analyze(idx)
    out_ref = jax.new_ref(mem)
    apply_k(wrow, wpos, counts, val, out_ref)
    return out_ref[...]


# speedup vs baseline: 1.0673x; 1.0383x over previous
import functools

import jax
import jax.numpy as jnp
from jax import lax
from jax.experimental import pallas as pl
from jax.experimental.pallas import tpu as pltpu
from jax.experimental.pallas import tpu_sc as plsc

NC = 2
NS = 16
NW = NC * NS
L = 16

CHUNK = 128


def _sc_body(M, B, R, idx_hbm, val_hbm, out_hbm,
             idx_v, row_buf, pos_buf, table, rot16, keep16, rows0, rows1,
             sem_g0, sem_g1, sem_s0, sem_s1):
    wid = lax.axis_index("s") * NC + lax.axis_index("c")
    lo = wid * R
    hi = jnp.minimum(lo + R, M)
    iota = lax.iota(jnp.int32, L)

    pltpu.sync_copy(idx_hbm, idx_v)

    def one_chunk(base, cursor):
        x = plsc.load_gather(idx_v, [base + iota])
        pos = base + iota
        m = (x >= lo) & (x < hi)
        mi = m.astype(jnp.int32)
        cnt = jnp.sum(mi)

        @pl.when(cnt > 0)
        def _():
            dest = cursor + jnp.cumsum(mi) - 1
            plsc.store_scatter(
                row_buf, [dest >> 7, dest & (CHUNK - 1)], x, mask=m)
            plsc.store_scatter(
                pos_buf, [dest >> 7, dest & (CHUNK - 1)], pos, mask=m)
            keep16[...] = mi

            @pl.when(cnt > 1)
            def _():
                rot16[...] = jnp.where(m, x, -1)
                dup = jnp.zeros((L,), jnp.bool_)
                for r in range(1, L):
                    y = plsc.load_gather(rot16, [(iota + r) & (L - 1)])
                    later = (iota + r) < L
                    dup = dup | (m & later & (y == x))
                keep16[...] = jnp.where(dup, 0, mi)

            keep = keep16[...] > 0
            plsc.store_scatter(table, [x - lo], pos, mask=keep)

        return cursor + cnt

    def scan_body(i, cursor):
        cursor = one_chunk(i * 2 * L, cursor)
        cursor = one_chunk(i * 2 * L + L, cursor)
        return cursor

    n_cand = lax.fori_loop(0, B // (2 * L), scan_body, jnp.int32(0))

    def win_body(c, carry):
        wcur, pad_row, pad_pos = carry
        fl = c * L + iota
        valid = fl < n_cand
        fb, fc = fl >> 7, fl & (CHUNK - 1)
        x = plsc.load_gather(row_buf, [fb, fc], mask=valid)
        p = plsc.load_gather(pos_buf, [fb, fc], mask=valid)
        w = plsc.load_gather(table, [jnp.where(valid, x - lo, 0)], mask=valid)
        keep = valid & (w == p)
        ki = keep.astype(jnp.int32)
        kcnt = jnp.sum(ki)
        dest = wcur + jnp.cumsum(ki) - 1
        plsc.store_scatter(row_buf, [dest >> 7, dest & (CHUNK - 1)], x, mask=keep)
        plsc.store_scatter(pos_buf, [dest >> 7, dest & (CHUNK - 1)], p, mask=keep)
        lmax = jnp.max(jnp.where(keep, iota, -1))
        sel = keep & (iota == lmax)
        pr = jnp.max(jnp.where(sel, x, -1))
        pp = jnp.max(jnp.where(sel, p, -1))
        pad_row = jnp.where(kcnt > 0, pr, pad_row)
        pad_pos = jnp.where(kcnt > 0, pp, pad_pos)
        return wcur + kcnt, pad_row, pad_pos

    n_win, pad_row, pad_pos = lax.fori_loop(
        0, pl.cdiv(n_cand, L), win_body,
        (jnp.int32(0), jnp.int32(0), jnp.int32(0)))

    n_tot = pl.cdiv(n_win, CHUNK) * CHUNK

    def pad_body(c, _):
        e = n_win + c * L + iota
        mm = e < n_tot
        plsc.store_scatter(
            row_buf, [e >> 7, e & (CHUNK - 1)],
            jnp.full((L,), pad_row, jnp.int32), mask=mm)
        plsc.store_scatter(
            pos_buf, [e >> 7, e & (CHUNK - 1)],
            jnp.full((L,), pad_pos, jnp.int32), mask=mm)
        return 0

    lax.fori_loop(0, pl.cdiv(n_tot - n_win, L), pad_body, 0)

    nb = n_tot // CHUNK

    def gather(b, rows, sem):
        pltpu.make_async_copy(
            val_hbm.at[pos_buf.at[b]], rows, sem).start()

    @pl.when(nb > 0)
    def _():
        gather(0, rows0, sem_g0)

    def pair_body(i, _):
        b = i * 2
        pltpu.make_async_copy(
            val_hbm.at[pos_buf.at[b]], rows0, sem_g0).wait()
        s0 = pltpu.make_async_copy(
            rows0, out_hbm.at[row_buf.at[b]], sem_s0)
        s0.start()

        @pl.when(b + 1 < nb)
        def _():
            gather(b + 1, rows1, sem_g1)

        s0.wait()

        @pl.when(b + 1 < nb)
        def _():
            pltpu.make_async_copy(
                val_hbm.at[pos_buf.at[b + 1]], rows1, sem_g1).wait()
            s1 = pltpu.make_async_copy(
                rows1, out_hbm.at[row_buf.at[b + 1]], sem_s1)
            s1.start()

            @pl.when(b + 2 < nb)
            def _():
                gather(b + 2, rows0, sem_g0)

            s1.wait()

        return 0

    lax.fori_loop(0, pl.cdiv(nb, 2), pair_body, 0)


def kernel(mem, idx, val):
    M, D = mem.shape
    B, _ = val.shape
    assert B % (2 * L) == 0
    R = (M + NW - 1) // NW
    assert 0 < M - (NW - 1) * R <= R
    NB_MAX = B // CHUNK

    mesh = plsc.VectorSubcoreMesh(
        core_axis_name="c", subcore_axis_name="s", num_cores=NC)

    sc = pl.kernel(
        functools.partial(_sc_body, M, B, R),
        out_type=(),
        mesh=mesh,
        compiler_params=pltpu.CompilerParams(needs_layout_passes=False),
        cost_estimate=pl.CostEstimate(
            flops=0, bytes_accessed=40_000_000, transcendentals=0),
        scratch_types=[
            pltpu.VMEM((B,), jnp.int32),
            pltpu.VMEM((NB_MAX, CHUNK), jnp.int32),
            pltpu.VMEM((NB_MAX, CHUNK), jnp.int32),
            pltpu.VMEM((R,), jnp.int32),
            pltpu.VMEM((L,), jnp.int32),
            pltpu.VMEM((L,), jnp.int32),
            pltpu.VMEM((CHUNK, D), jnp.float32),
            pltpu.VMEM((CHUNK, D), jnp.float32),
            pltpu.SemaphoreType.DMA,
            pltpu.SemaphoreType.DMA,
            pltpu.SemaphoreType.DMA,
            pltpu.SemaphoreType.DMA,
        ],
    )

    out_ref = jax.new_ref(mem)
    sc(idx, val, out_ref)
    return out_ref[...]
